# baseline (speedup 1.0000x reference)
import functools
import math

import jax
import jax.numpy as jnp
from jax.experimental import pallas as pl
from jax.experimental.pallas import tpu as pltpu

_S = 32.0
_M = 0.5
_COS_M = math.cos(_M)
_SIN_M = math.sin(_M)
_TH = math.cos(math.pi - _M)
_MM = math.sin(math.pi - _M) * _M
_NORM_EPS = 1e-12


def _pool_body(ids_ref, table_hbm, img_ref, cnn_w_ref,
               tsum_ref, x_ref,
               table_vmem, dma_sem,
               *, half, seq, rows_step, nchan, pix, img_every):
    c = pl.program_id(0)
    k = pl.program_id(1)
    hs = table_vmem.shape[2]

    @pl.when(k == 0)
    def _load_table():
        cp = pltpu.make_async_copy(
            table_hbm.at[pl.ds(c * half, half)],
            table_vmem.at[pl.ds(0, half), 0],
            dma_sem)
        cp.start()
        table_vmem[half] = jnp.zeros((1, hs), jnp.float32)
        cp.wait()

    @pl.when(k > 0)
    def _work():
        for j in range(rows_step):
            base = (k - 1) * rows_step * seq + j * seq
            accs = [jnp.zeros((1, hs), jnp.float32) for _ in range(2)]
            for s in range(0, seq, 2):
                for u in range(2):
                    accs[u] = accs[u] + table_vmem[ids_ref[0, 0, base + s + u]]
            tsum_ref[(k - 1) * rows_step + j] = accs[0] + accs[1]

        @pl.when((k - 1) % img_every == img_every - 1)
        def _project():
            gap = jnp.sum(img_ref[...], axis=(2, 3))
            x_ref[...] = jnp.dot(gap, cnn_w_ref[...],
                                 preferred_element_type=jnp.float32)


def _t2red_body(g_ref, out_ref):
    out_ref[0] = jnp.sum(g_ref[...], axis=0, keepdims=True)


def _head_body(label_ref, x_ref, t1_ref, t2_ref,
               cs_ref, ch_ref, bs_ref, bh_ref,
               wi_ref, w1_ref, w2_ref, b_ref, wnt_ref,
               logits_ref, ret_ref, fn_ref, *, tn):
    ci = pl.program_id(1)

    @pl.when(ci == 0)
    def _embed():
        xf = x_ref[...] * cs_ref[...] + ch_ref[...]
        t1 = t1_ref[...] * bs_ref[...] + bh_ref[...]
        t2 = t2_ref[...] * bs_ref[...] + bh_ref[...]
        acc = jnp.dot(xf.astype(jnp.bfloat16), wi_ref[...],
                      preferred_element_type=jnp.float32)
        acc = acc + jnp.dot(t1.astype(jnp.bfloat16), w1_ref[...],
                            preferred_element_type=jnp.float32)
        acc = acc + jnp.dot(t2.astype(jnp.bfloat16), w2_ref[...],
                            preferred_element_type=jnp.float32)
        acc = acc + b_ref[...]
        ret_ref[...] = acc
        inv = jax.lax.rsqrt(jnp.sum(acc * acc, axis=1, keepdims=True) + _NORM_EPS)
        fn_ref[...] = (acc * inv).astype(jnp.bfloat16)

    cos = jnp.dot(fn_ref[...], wnt_ref[...], preferred_element_type=jnp.float32)
    sin = jnp.sqrt(jnp.clip(1.0 - cos * cos, 0.0, 1.0))
    phi = jnp.where(cos > _TH, cos * _COS_M - sin * _SIN_M, cos - _MM)
    cls = ci * tn + jax.lax.broadcasted_iota(jnp.int32, cos.shape, 1)
    logits_ref[...] = jnp.where(cls == label_ref[...], phi, cos) * _S


def kernel(X_image, input_ids, attention_mask, input_ids2, attention_mask2,
           label, cnn_w, bert_emb, cnn_scale, cnn_shift, bert_scale,
           bert_shift, w_img, w_t1, w_t2, b_fold, arc_wnt_pad):
    del attention_mask, attention_mask2

    B = X_image.shape[0]
    nchan = X_image.shape[1]
    pix = X_image.shape[2] * X_image.shape[3]
    seq = input_ids.shape[1]
    V, hs = bert_emb.shape
    cf = cnn_w.shape[1]
    o = b_fold.shape[1]
    C = arc_wnt_pad.shape[1]
    half = V // 2

    nstep = 128 if B >= 256 else 8
    rows_step = B // nstep
    img_every = 8
    x_chunks = B // (2 * img_every)
    ni = B * seq

    ids_flat = input_ids.reshape(-1).astype(jnp.int32)
    rel = ids_flat[None, :] - jnp.array([[0], [half]], jnp.int32)
    ids_core = jnp.where((rel >= 0) & (rel < half), rel, half).reshape(2, 1, ni)
    t2g = jnp.take(bert_emb, input_ids2.reshape(-1), axis=0)

    cnn_w_scaled = cnn_w * (1.0 / pix)

    pool_grid = (2, nstep + 1)
    img_blk = lambda c, k: c * x_chunks + jnp.clip(
        (k - 1) // img_every, 0, x_chunks - 1)
    img_idx = lambda c, k: (img_blk(c, k), 0, 0, 0)
    x_blk = lambda c, k: (img_blk(c, k), 0)

    tsum, x = pl.pallas_call(
        functools.partial(_pool_body, half=half, seq=seq,
                          rows_step=rows_step, nchan=nchan, pix=pix,
                          img_every=img_every),
        grid=pool_grid,
        in_specs=[
            pl.BlockSpec((1, 1, ni), lambda c, k: (c, 0, 0),
                         memory_space=pltpu.SMEM),
            pl.BlockSpec(memory_space=pl.ANY),
            pl.BlockSpec((img_every, nchan, X_image.shape[2], X_image.shape[3]),
                         img_idx),
            pl.BlockSpec((nchan, cf), lambda c, k: (0, 0)),
        ],
        out_shape=(jax.ShapeDtypeStruct((2 * B, 1, hs), jnp.float32),
                   jax.ShapeDtypeStruct((B, cf), jnp.float32)),
        out_specs=(
            pl.BlockSpec((B, 1, hs), lambda c, k: (c, 0, 0)),
            pl.BlockSpec((img_every, cf), x_blk),
        ),
        scratch_shapes=[
            pltpu.VMEM((half + 1, 1, hs), jnp.float32),
            pltpu.SemaphoreType.DMA,
        ],
        compiler_params=pltpu.CompilerParams(
            dimension_semantics=("parallel", "arbitrary"),
            vmem_limit_bytes=61_000_000),
        cost_estimate=pl.CostEstimate(
            flops=2 * B * nchan * cf,
            transcendentals=0,
            bytes_accessed=4 * (V * hs + B * nchan * pix + 4 * B * hs)),
    )(ids_core, bert_emb, X_image, cnn_w_scaled)

    t1 = tsum[:B, 0, :] + tsum[B:, 0, :]
    t2red = pl.pallas_call(
        _t2red_body,
        grid=(B,),
        in_specs=[pl.BlockSpec((seq, hs), lambda i: (i, 0))],
        out_shape=jax.ShapeDtypeStruct((B, 1, hs), jnp.float32),
        out_specs=pl.BlockSpec((1, 1, hs), lambda i: (i, 0, 0)),
        compiler_params=pltpu.CompilerParams(
            dimension_semantics=("parallel",)),
    )(t2g)
    t2 = t2red[:, 0, :]
    bs_scaled = bert_scale * (1.0 / seq)

    tm = min(128, B)
    nb = B // tm
    tn = min(1024, C)
    nc = C // tn

    label_col = label.astype(jnp.int32).reshape(B, 1)

    blk_b = lambda bi, ci: (bi, 0)
    blk_0 = lambda bi, ci: (0, 0)
    blk_c = lambda bi, ci: (0, ci)
    blk_bc = lambda bi, ci: (bi, ci)

    in_specs = [
        pl.BlockSpec((tm, 1), blk_b),
        pl.BlockSpec((tm, cf), blk_b),
        pl.BlockSpec((tm, hs), blk_b),
        pl.BlockSpec((tm, hs), blk_b),
        pl.BlockSpec((1, cf), blk_0),
        pl.BlockSpec((1, cf), blk_0),
        pl.BlockSpec((1, hs), blk_0),
        pl.BlockSpec((1, hs), blk_0),
        pl.BlockSpec((cf, o), blk_0),
        pl.BlockSpec((hs, o), blk_0),
        pl.BlockSpec((hs, o), blk_0),
        pl.BlockSpec((1, o), blk_0),
        pl.BlockSpec((o, tn), blk_c),
    ]
    out_specs = (
        pl.BlockSpec((tm, tn), blk_bc),
        pl.BlockSpec((tm, o), blk_b),
    )
    logits, ret = pl.pallas_call(
        functools.partial(_head_body, tn=tn),
        grid=(nb, nc),
        out_shape=(jax.ShapeDtypeStruct((B, C), jnp.float32),
                   jax.ShapeDtypeStruct((B, o), jnp.float32)),
        in_specs=in_specs,
        out_specs=out_specs,
        scratch_shapes=[pltpu.VMEM((tm, o), jnp.bfloat16)],
        compiler_params=pltpu.CompilerParams(
            dimension_semantics=("parallel", "arbitrary"),
            vmem_limit_bytes=48 * 1024 * 1024),
    )(label_col, x, t1, t2, cnn_scale, cnn_shift, bs_scaled, bert_shift,
      w_img, w_t1, w_t2, b_fold, arc_wnt_pad)
    return logits, ret


# speedup vs baseline: 1.7584x; 1.7584x over previous
import functools
import math

import jax
import jax.numpy as jnp
from jax.experimental import pallas as pl
from jax.experimental.pallas import tpu as pltpu

_S = 32.0
_M = 0.5
_COS_M = math.cos(_M)
_SIN_M = math.sin(_M)
_TH = math.cos(math.pi - _M)
_MM = math.sin(math.pi - _M) * _M
_NORM_EPS = 1e-12


def _pool_body(ids_ref, table_hbm, img_ref, cnn_w_ref,
               tsum_ref, x_ref,
               table_vmem, dma_sem,
               *, half, seq, rows_step, nchan, pix, img_every):
    c = pl.program_id(0)
    k = pl.program_id(1)
    hs = table_vmem.shape[2]

    @pl.when(k == 0)
    def _load_table():
        cp = pltpu.make_async_copy(
            table_hbm.at[pl.ds(c * half, half)],
            table_vmem.at[pl.ds(0, half), 0],
            dma_sem)
        cp.start()
        table_vmem[half] = jnp.zeros((1, hs), jnp.float32)
        cp.wait()

    @pl.when(k > 0)
    def _work():
        for j in range(rows_step):
            base = (k - 1) * rows_step * seq + j * seq
            accs = [jnp.zeros((1, hs), jnp.float32) for _ in range(2)]
            for s in range(0, seq, 2):
                for u in range(2):
                    accs[u] = accs[u] + table_vmem[ids_ref[0, 0, base + s + u]]
            tsum_ref[(k - 1) * rows_step + j] = accs[0] + accs[1]

        @pl.when((k - 1) % img_every == img_every - 1)
        def _project():
            gap = jnp.sum(img_ref[...], axis=(2, 3))
            x_ref[...] = jnp.dot(gap, cnn_w_ref[...],
                                 preferred_element_type=jnp.float32)


def _head_body(label_ref, x_ref, t1_ref, t2_ref,
               cs_ref, ch_ref, bs_ref, bh_ref,
               wi_ref, w1_ref, w2_ref, b_ref, wnt_ref,
               logits_ref, ret_ref, fn_ref, *, tn):
    ci = pl.program_id(1)

    @pl.when(ci == 0)
    def _embed():
        xf = x_ref[...] * cs_ref[...] + ch_ref[...]
        t1 = t1_ref[...] * bs_ref[...] + bh_ref[...]
        t2 = t2_ref[...] * bs_ref[...] + bh_ref[...]
        acc = jnp.dot(xf.astype(jnp.bfloat16), wi_ref[...],
                      preferred_element_type=jnp.float32)
        acc = acc + jnp.dot(t1.astype(jnp.bfloat16), w1_ref[...],
                            preferred_element_type=jnp.float32)
        acc = acc + jnp.dot(t2.astype(jnp.bfloat16), w2_ref[...],
                            preferred_element_type=jnp.float32)
        acc = acc + b_ref[...]
        ret_ref[...] = acc
        inv = jax.lax.rsqrt(jnp.sum(acc * acc, axis=1, keepdims=True) + _NORM_EPS)
        fn_ref[...] = (acc * inv).astype(jnp.bfloat16)

    cos = jnp.dot(fn_ref[...], wnt_ref[...], preferred_element_type=jnp.float32)
    sin = jnp.sqrt(jnp.clip(1.0 - cos * cos, 0.0, 1.0))
    phi = jnp.where(cos > _TH, cos * _COS_M - sin * _SIN_M, cos - _MM)
    cls = ci * tn + jax.lax.broadcasted_iota(jnp.int32, cos.shape, 1)
    logits_ref[...] = jnp.where(cls == label_ref[...], phi, cos) * _S


def kernel(X_image, input_ids, attention_mask, input_ids2, attention_mask2,
           label, cnn_w, bert_emb, cnn_scale, cnn_shift, bert_scale,
           bert_shift, w_img, w_t1, w_t2, b_fold, arc_wnt_pad):
    del attention_mask, attention_mask2

    B = X_image.shape[0]
    nchan = X_image.shape[1]
    pix = X_image.shape[2] * X_image.shape[3]
    seq = input_ids.shape[1]
    V, hs = bert_emb.shape
    cf = cnn_w.shape[1]
    o = b_fold.shape[1]
    C = arc_wnt_pad.shape[1]
    half = V // 2

    nstep = 128 if B >= 256 else 8
    rows_step = (2 * B) // nstep
    img_every = 8
    x_chunks = B // (2 * img_every)
    ni = 2 * B * seq

    ids_flat = jnp.concatenate(
        [input_ids.reshape(-1), input_ids2.reshape(-1)]).astype(jnp.int32)
    rel = ids_flat[None, :] - jnp.array([[0], [half]], jnp.int32)
    ids_core = jnp.where((rel >= 0) & (rel < half), rel, half).reshape(2, 1, ni)

    cnn_w_scaled = cnn_w * (1.0 / pix)

    pool_grid = (2, nstep + 1)
    img_blk = lambda c, k: c * x_chunks + jnp.clip(
        (k - 1) // img_every, 0, x_chunks - 1)
    img_idx = lambda c, k: (img_blk(c, k), 0, 0, 0)
    x_blk = lambda c, k: (img_blk(c, k), 0)

    tsum, x = pl.pallas_call(
        functools.partial(_pool_body, half=half, seq=seq,
                          rows_step=rows_step, nchan=nchan, pix=pix,
                          img_every=img_every),
        grid=pool_grid,
        in_specs=[
            pl.BlockSpec((1, 1, ni), lambda c, k: (c, 0, 0),
                         memory_space=pltpu.SMEM),
            pl.BlockSpec(memory_space=pl.ANY),
            pl.BlockSpec((img_every, nchan, X_image.shape[2], X_image.shape[3]),
                         img_idx),
            pl.BlockSpec((nchan, cf), lambda c, k: (0, 0)),
        ],
        out_shape=(jax.ShapeDtypeStruct((4 * B, 1, hs), jnp.float32),
                   jax.ShapeDtypeStruct((B, cf), jnp.float32)),
        out_specs=(
            pl.BlockSpec((2 * B, 1, hs), lambda c, k: (c, 0, 0)),
            pl.BlockSpec((img_every, cf), x_blk),
        ),
        scratch_shapes=[
            pltpu.VMEM((half + 1, 1, hs), jnp.float32),
            pltpu.SemaphoreType.DMA,
        ],
        compiler_params=pltpu.CompilerParams(
            dimension_semantics=("parallel", "arbitrary"),
            vmem_limit_bytes=61_000_000),
        cost_estimate=pl.CostEstimate(
            flops=2 * B * nchan * cf,
            transcendentals=0,
            bytes_accessed=4 * (V * hs + B * nchan * pix + 4 * B * hs)),
    )(ids_core, bert_emb, X_image, cnn_w_scaled)

    tsum2d = tsum[:2 * B, 0, :] + tsum[2 * B:, 0, :]
    t1 = tsum2d[:B]
    t2 = tsum2d[B:]
    bs_scaled = bert_scale * (1.0 / seq)

    tm = min(128, B)
    nb = B // tm
    tn = min(1024, C)
    nc = C // tn

    label_col = label.astype(jnp.int32).reshape(B, 1)

    blk_b = lambda bi, ci: (bi, 0)
    blk_0 = lambda bi, ci: (0, 0)
    blk_c = lambda bi, ci: (0, ci)
    blk_bc = lambda bi, ci: (bi, ci)

    in_specs = [
        pl.BlockSpec((tm, 1), blk_b),
        pl.BlockSpec((tm, cf), blk_b),
        pl.BlockSpec((tm, hs), blk_b),
        pl.BlockSpec((tm, hs), blk_b),
        pl.BlockSpec((1, cf), blk_0),
        pl.BlockSpec((1, cf), blk_0),
        pl.BlockSpec((1, hs), blk_0),
        pl.BlockSpec((1, hs), blk_0),
        pl.BlockSpec((cf, o), blk_0),
        pl.BlockSpec((hs, o), blk_0),
        pl.BlockSpec((hs, o), blk_0),
        pl.BlockSpec((1, o), blk_0),
        pl.BlockSpec((o, tn), blk_c),
    ]
    out_specs = (
        pl.BlockSpec((tm, tn), blk_bc),
        pl.BlockSpec((tm, o), blk_b),
    )
    logits, ret = pl.pallas_call(
        functools.partial(_head_body, tn=tn),
        grid=(nb, nc),
        out_shape=(jax.ShapeDtypeStruct((B, C), jnp.float32),
                   jax.ShapeDtypeStruct((B, o), jnp.float32)),
        in_specs=in_specs,
        out_specs=out_specs,
        scratch_shapes=[pltpu.VMEM((tm, o), jnp.bfloat16)],
        compiler_params=pltpu.CompilerParams(
            dimension_semantics=("parallel", "arbitrary"),
            vmem_limit_bytes=48 * 1024 * 1024),
    )(label_col, x, t1, t2, cnn_scale, cnn_shift, bs_scaled, bert_shift,
      w_img, w_t1, w_t2, b_fold, arc_wnt_pad)
    return logits, ret


# speedup vs baseline: 1.7786x; 1.0115x over previous
import functools
import math

import jax
import jax.numpy as jnp
from jax.experimental import pallas as pl
from jax.experimental.pallas import tpu as pltpu

_S = 32.0
_M = 0.5
_COS_M = math.cos(_M)
_SIN_M = math.sin(_M)
_TH = math.cos(math.pi - _M)
_MM = math.sin(math.pi - _M) * _M
_NORM_EPS = 1e-12


def _pool_body(ids_ref, table_hbm, img_ref, cnn_w_ref,
               tsum_ref, x_ref,
               table_vmem, dma_sem,
               *, half, seq, rows_step, nchan, pix, img_every):
    c = pl.program_id(0)
    k = pl.program_id(1)
    hs = table_vmem.shape[2]

    @pl.when(k == 0)
    def _load_table():
        nchunk = 4
        chunk = half // nchunk

        def cp(i):
            return pltpu.make_async_copy(
                table_hbm.at[pl.ds(c * half + i * chunk, chunk)],
                table_vmem.at[pl.ds(i * chunk, chunk), 0],
                dma_sem.at[i])

        for i in range(nchunk):
            cp(i).start()
        table_vmem[half] = jnp.zeros((1, hs), jnp.float32)
        for i in range(nchunk):
            cp(i).wait()

    @pl.when(k > 0)
    def _work():
        for j in range(rows_step):
            base = (k - 1) * rows_step * seq + j * seq
            accs = [jnp.zeros((1, hs), jnp.float32) for _ in range(2)]
            for s in range(0, seq, 2):
                for u in range(2):
                    accs[u] = accs[u] + table_vmem[ids_ref[0, 0, base + s + u]]
            tsum_ref[(k - 1) * rows_step + j] = accs[0] + accs[1]

        @pl.when((k - 1) % img_every == img_every - 1)
        def _project():
            gap = jnp.sum(img_ref[...], axis=(2, 3))
            x_ref[...] = jnp.dot(gap, cnn_w_ref[...],
                                 preferred_element_type=jnp.float32)


def _head_body(label_ref, x_ref, t1_ref, t2_ref,
               cs_ref, ch_ref, bs_ref, bh_ref,
               wi_ref, w1_ref, w2_ref, b_ref, wnt_ref,
               logits_ref, ret_ref, fn_ref, *, tn):
    ci = pl.program_id(1)

    @pl.when(ci == 0)
    def _embed():
        xf = x_ref[...] * cs_ref[...] + ch_ref[...]
        t1 = t1_ref[...] * bs_ref[...] + bh_ref[...]
        t2 = t2_ref[...] * bs_ref[...] + bh_ref[...]
        acc = jnp.dot(xf.astype(jnp.bfloat16), wi_ref[...],
                      preferred_element_type=jnp.float32)
        acc = acc + jnp.dot(t1.astype(jnp.bfloat16), w1_ref[...],
                            preferred_element_type=jnp.float32)
        acc = acc + jnp.dot(t2.astype(jnp.bfloat16), w2_ref[...],
                            preferred_element_type=jnp.float32)
        acc = acc + b_ref[...]
        ret_ref[...] = acc
        inv = jax.lax.rsqrt(jnp.sum(acc * acc, axis=1, keepdims=True) + _NORM_EPS)
        fn_ref[...] = (acc * inv).astype(jnp.bfloat16)

    cos = jnp.dot(fn_ref[...], wnt_ref[...], preferred_element_type=jnp.float32)
    sin = jnp.sqrt(jnp.clip(1.0 - cos * cos, 0.0, 1.0))
    phi = jnp.where(cos > _TH, cos * _COS_M - sin * _SIN_M, cos - _MM)
    cls = ci * tn + jax.lax.broadcasted_iota(jnp.int32, cos.shape, 1)
    logits_ref[...] = jnp.where(cls == label_ref[...], phi, cos) * _S


def kernel(X_image, input_ids, attention_mask, input_ids2, attention_mask2,
           label, cnn_w, bert_emb, cnn_scale, cnn_shift, bert_scale,
           bert_shift, w_img, w_t1, w_t2, b_fold, arc_wnt_pad):
    del attention_mask, attention_mask2

    B = X_image.shape[0]
    nchan = X_image.shape[1]
    pix = X_image.shape[2] * X_image.shape[3]
    seq = input_ids.shape[1]
    V, hs = bert_emb.shape
    cf = cnn_w.shape[1]
    o = b_fold.shape[1]
    C = arc_wnt_pad.shape[1]
    half = V // 2

    nstep = 128 if B >= 256 else 8
    rows_step = (2 * B) // nstep
    img_every = 8
    x_chunks = B // (2 * img_every)
    ni = 2 * B * seq

    ids_flat = jnp.concatenate(
        [input_ids.reshape(-1), input_ids2.reshape(-1)]).astype(jnp.int32)
    rel = ids_flat[None, :] - jnp.array([[0], [half]], jnp.int32)
    ids_core = jnp.where((rel >= 0) & (rel < half), rel, half).reshape(2, 1, ni)

    cnn_w_scaled = cnn_w * (1.0 / pix)

    pool_grid = (2, nstep + 1)
    img_blk = lambda c, k: c * x_chunks + jnp.clip(
        (k - 1) // img_every, 0, x_chunks - 1)
    img_idx = lambda c, k: (img_blk(c, k), 0, 0, 0)
    x_blk = lambda c, k: (img_blk(c, k), 0)

    tsum, x = pl.pallas_call(
        functools.partial(_pool_body, half=half, seq=seq,
                          rows_step=rows_step, nchan=nchan, pix=pix,
                          img_every=img_every),
        grid=pool_grid,
        in_specs=[
            pl.BlockSpec((1, 1, ni), lambda c, k: (c, 0, 0),
                         memory_space=pltpu.SMEM),
            pl.BlockSpec(memory_space=pl.ANY),
            pl.BlockSpec((img_every, nchan, X_image.shape[2], X_image.shape[3]),
                         img_idx),
            pl.BlockSpec((nchan, cf), lambda c, k: (0, 0)),
        ],
        out_shape=(jax.ShapeDtypeStruct((4 * B, 1, hs), jnp.float32),
                   jax.ShapeDtypeStruct((B, cf), jnp.float32)),
        out_specs=(
            pl.BlockSpec((2 * B, 1, hs), lambda c, k: (c, 0, 0)),
            pl.BlockSpec((img_every, cf), x_blk),
        ),
        scratch_shapes=[
            pltpu.VMEM((half + 1, 1, hs), jnp.float32),
            pltpu.SemaphoreType.DMA((4,)),
        ],
        compiler_params=pltpu.CompilerParams(
            dimension_semantics=("parallel", "arbitrary"),
            vmem_limit_bytes=61_000_000),
        cost_estimate=pl.CostEstimate(
            flops=2 * B * nchan * cf,
            transcendentals=0,
            bytes_accessed=4 * (V * hs + B * nchan * pix + 4 * B * hs)),
    )(ids_core, bert_emb, X_image, cnn_w_scaled)

    tsum2d = tsum[:2 * B, 0, :] + tsum[2 * B:, 0, :]
    t1 = tsum2d[:B]
    t2 = tsum2d[B:]
    bs_scaled = bert_scale * (1.0 / seq)

    tm = min(128, B)
    nb = B // tm
    tn = min(1024, C)
    nc = C // tn

    label_col = label.astype(jnp.int32).reshape(B, 1)

    blk_b = lambda bi, ci: (bi, 0)
    blk_0 = lambda bi, ci: (0, 0)
    blk_c = lambda bi, ci: (0, ci)
    blk_bc = lambda bi, ci: (bi, ci)

    in_specs = [
        pl.BlockSpec((tm, 1), blk_b),
        pl.BlockSpec((tm, cf), blk_b),
        pl.BlockSpec((tm, hs), blk_b),
        pl.BlockSpec((tm, hs), blk_b),
        pl.BlockSpec((1, cf), blk_0),
        pl.BlockSpec((1, cf), blk_0),
        pl.BlockSpec((1, hs), blk_0),
        pl.BlockSpec((1, hs), blk_0),
        pl.BlockSpec((cf, o), blk_0),
        pl.BlockSpec((hs, o), blk_0),
        pl.BlockSpec((hs, o), blk_0),
        pl.BlockSpec((1, o), blk_0),
        pl.BlockSpec((o, tn), blk_c),
    ]
    out_specs = (
        pl.BlockSpec((tm, tn), blk_bc),
        pl.BlockSpec((tm, o), blk_b),
    )
    logits, ret = pl.pallas_call(
        functools.partial(_head_body, tn=tn),
        grid=(nb, nc),
        out_shape=(jax.ShapeDtypeStruct((B, C), jnp.float32),
                   jax.ShapeDtypeStruct((B, o), jnp.float32)),
        in_specs=in_specs,
        out_specs=out_specs,
        scratch_shapes=[pltpu.VMEM((tm, o), jnp.bfloat16)],
        compiler_params=pltpu.CompilerParams(
            dimension_semantics=("parallel", "arbitrary"),
            vmem_limit_bytes=48 * 1024 * 1024),
    )(label_col, x, t1, t2, cnn_scale, cnn_shift, bs_scaled, bert_shift,
      w_img, w_t1, w_t2, b_fold, arc_wnt_pad)
    return logits, ret
